# baseline (speedup 1.0000x reference)
import jax
import jax.numpy as jnp
from jax import lax
from jax.experimental import pallas as pl
from jax.experimental.pallas import tpu as pltpu
from jax.experimental.pallas import tpu_sc as plsc

_B, _S, _D = 2, 2048, 2048
_E, _R, _CH = 8, 16, 128
_NC = _S // _CH
_NCH = _B * _NC
_EP = 16
_H = _D // 2
_TAU = 0.7
_SCALE = 16.0 / _R
_NEG = -1e30


def _router_body(x_ref, w1_ref, b1_ref, w2_ref, b2_ref, out_ref):
    h = jnp.dot(x_ref[...], w1_ref[...], preferred_element_type=jnp.float32)
    h = jnp.maximum(h + b1_ref[...], 0.0)
    logits = jnp.dot(h, w2_ref[...], preferred_element_type=jnp.float32)
    logits = logits + b2_ref[...]
    out_ref[...] = jnp.mean(logits, axis=0, keepdims=True)[None]


def _chunk_logits(x2d, Wr1, br1, Wr2p, br2p):
    return pl.pallas_call(
        _router_body,
        grid=(_NCH,),
        in_specs=[
            pl.BlockSpec((_CH, _D), lambda i: (i, 0)),
            pl.BlockSpec((_D, _H), lambda i: (0, 0)),
            pl.BlockSpec((1, _H), lambda i: (0, 0)),
            pl.BlockSpec((_H, _EP), lambda i: (0, 0)),
            pl.BlockSpec((1, _EP), lambda i: (0, 0)),
        ],
        out_specs=pl.BlockSpec((1, 1, _EP), lambda i: (i, 0, 0)),
        out_shape=jax.ShapeDtypeStruct((_NCH, 1, _EP), jnp.float32),
    )(x2d, Wr1, br1, Wr2p, br2p)


def _take16(v, idx):
    return jnp.take_along_axis(v, idx, axis=0, mode="promise_in_bounds")


def _route_sc_body(cl_hbm, idx_hbm, cl_v, idx_v, sem):
    del sem
    wid = lax.axis_index("s") * 2 + lax.axis_index("c")

    @pl.when(wid == 0)
    def _():
        pltpu.sync_copy(cl_hbm, cl_v)
        lane = lax.iota(jnp.int32, 16)
        big = jnp.full((16,), 16, jnp.int32)
        for b in range(_B):
            prev = jnp.zeros((16,), jnp.int32)
            acc = jnp.zeros((16,), jnp.int32)
            for c in range(_NC):
                cl = cl_v[b * _NC + c]
                m = cl
                for k in (1, 2, 4, 8):
                    m = jnp.maximum(m, _take16(m, lane ^ k))
                cand = jnp.where(cl == m, lane, big)
                ce = cand
                for k in (1, 2, 4, 8):
                    ce = jnp.minimum(ce, _take16(ce, lane ^ k))
                if c > 0:
                    d_old = _take16(cl, prev)
                    ce = jnp.where((m - d_old) > _TAU, ce, prev)
                prev = ce
                acc = jnp.where(lane == c, ce, acc)
            idx_v[b] = acc
        pltpu.sync_copy(idx_v, idx_hbm)


def _route_sc(cl2d):
    mesh = plsc.VectorSubcoreMesh(core_axis_name="c", subcore_axis_name="s")
    return pl.kernel(
        _route_sc_body,
        out_type=jax.ShapeDtypeStruct((_B, _NC), jnp.int32),
        mesh=mesh,
        scratch_types=[
            pltpu.VMEM((_NCH, _EP), jnp.float32),
            pltpu.VMEM((_B, _NC), jnp.int32),
            pltpu.SemaphoreType.DMA,
        ],
    )(cl2d)


def _main_body(idx_sref, x_ref, wt_ref, b_ref, a_ref, bm_ref, out_ref, rw_ref):
    i = pl.program_id(0)
    e = idx_sref[i]
    xb = x_ref[...]
    base = jnp.dot(xb.astype(jnp.bfloat16), wt_ref[...],
                   preferred_element_type=jnp.float32)
    a_e = a_ref[e]
    ax = lax.dot_general(xb, a_e, (((1,), (1,)), ((), ())),
                         preferred_element_type=jnp.float32)
    routed = jnp.dot(ax, bm_ref[e], preferred_element_type=jnp.float32)
    out_ref[...] = base + routed * _SCALE + b_ref[0:1, :]
    col = lax.broadcasted_iota(jnp.int32, (_CH, _E), 1)
    rw_ref[...] = (col == e).astype(jnp.float32)


def _main(idx_flat, x2d, Wt_bf, b8, At, Bm):
    grid_spec = pltpu.PrefetchScalarGridSpec(
        num_scalar_prefetch=1,
        grid=(_NCH,),
        in_specs=[
            pl.BlockSpec((_CH, _D), lambda i, s: (i, 0)),
            pl.BlockSpec((_D, _D), lambda i, s: (0, 0)),
            pl.BlockSpec((8, _D), lambda i, s: (0, 0)),
            pl.BlockSpec((_E, _R, _D), lambda i, s: (0, 0, 0)),
            pl.BlockSpec((_E, _R, _D), lambda i, s: (0, 0, 0)),
        ],
        out_specs=[
            pl.BlockSpec((_CH, _D), lambda i, s: (i, 0)),
            pl.BlockSpec((_CH, _E), lambda i, s: (i, 0)),
        ],
    )
    return pl.pallas_call(
        _main_body,
        grid_spec=grid_spec,
        out_shape=[
            jax.ShapeDtypeStruct((_B * _S, _D), jnp.float32),
            jax.ShapeDtypeStruct((_B * _S, _E), jnp.float32),
        ],
    )(idx_flat, x2d, Wt_bf, b8, At, Bm)


def kernel(x, W_base, b_base, Wr1, br1, Wr2, br2, A, Bm):
    x2d = x.reshape(_B * _S, _D)
    Wr2p = jnp.pad(Wr2, ((0, 0), (0, _EP - _E)))
    br2p = jnp.concatenate(
        [br2, jnp.full((_EP - _E,), _NEG, jnp.float32)]).reshape(1, _EP)
    br1_2d = br1.reshape(1, _H)

    cl = _chunk_logits(x2d, Wr1, br1_2d, Wr2p, br2p)
    expert_idx = _route_sc(cl.reshape(_NCH, _EP))

    Wt_bf = W_base.T.astype(jnp.bfloat16)
    b8 = jnp.broadcast_to(b_base.reshape(1, _D), (8, _D))
    At = jnp.swapaxes(A, 1, 2)
    out2d, rw2d = _main(expert_idx.reshape(_NCH), x2d, Wt_bf, b8, At, Bm)

    output = out2d.reshape(_B, _S, _D)
    routing_weights = rw2d.reshape(_B, _S, _E)
    return output, routing_weights, expert_idx
